# Initial kernel scaffold; baseline (speedup 1.0000x reference)
import jax
import jax.numpy as jnp
from jax.experimental import pallas as pl


def kernel(table):
    raise NotImplementedError("write your pallas kernel here")


# speedup vs baseline: 1.3876x; 1.3876x over previous
import jax
import jax.numpy as jnp
from jax.experimental import pallas as pl


def _copy_block(in_ref, out_ref):
    out_ref[...] = in_ref[...]


def kernel(table):
    n, d = table.shape
    wide = table.reshape(n // 4, d * 4)
    rows = wide.shape[0]
    block = 2000
    out = pl.pallas_call(
        _copy_block,
        grid=(rows // block,),
        in_specs=[pl.BlockSpec((block, d * 4), lambda i: (i, 0))],
        out_specs=pl.BlockSpec((block, d * 4), lambda i: (i, 0)),
        out_shape=jax.ShapeDtypeStruct(wide.shape, wide.dtype),
    )(wide)
    return out.reshape(n, d)
